# baseline (speedup 1.0000x reference)
import jax
import jax.numpy as jnp
from jax import lax
from jax.experimental import pallas as pl
from jax.experimental.pallas import tpu as pltpu
from jax.experimental.pallas import tpu_sc as plsc

N = 100000
E = 6400000
NPAD = 100352
R = NPAD // 128
NC, NS = 2, 16
NW = NC * NS
SLICE = NPAD // NS
CHUNK = 2000
NQ_U = E // (NW * CHUNK) // 4
NQ_V = E // (NS * CHUNK) // 4
PW = E // NW
PWV = E // NS

_mesh = plsc.VectorSubcoreMesh(core_axis_name="c", subcore_axis_name="s")
_f32 = jnp.float32
_i32 = jnp.int32
_PC = 1920
_PIECES = [(o, min(_PC, SLICE - o)) for o in range(0, SLICE, _PC)]


def _fill(buf, nwords, value):
    vec = jnp.full((16,), value, _f32)

    def body(i, carry):
        buf[pl.ds(i * 16, 16)] = vec
        return carry

    lax.fori_loop(0, nwords // 16, body, 0)


def _zero_shared(sh_ref, off, valv):
    _fill(valv, CHUNK, 0.0)
    for o, ln in _PIECES:
        pltpu.sync_copy(valv.at[pl.ds(0, ln)], sh_ref.at[pl.ds(off + o, ln)])


def _export_shared(sh_ref, off, valv, out_slot):
    for o, ln in _PIECES:
        pltpu.sync_copy(sh_ref.at[pl.ds(off + o, ln)], valv.at[pl.ds(0, ln)])
        pltpu.sync_copy(valv.at[pl.ds(0, ln)], out_slot.at[pl.ds(off + o, ln)])


def _pump(src_hbm, dst_hbm, acc_sh, table, sv, dv, vv, se, sd, ss,
          base, nquads):
    def gather(srcv, valv):
        def g(i, carry):
            for u in range(5):
                o = i * 80 + u * 16
                idx = srcv[pl.ds(o, 16)]
                valv[pl.ds(o, 16)] = plsc.load_gather(table, [idx])
            return carry

        lax.fori_loop(0, CHUNK // 80, g, 0)

    def quad(t, carry):
        for half in range(2):
            b0, b1 = 2 * half, 2 * half + 1
            ca = base + (4 * t + 2 * half) * CHUNK
            cb = ca + CHUNK

            @pl.when(t > 0)
            def _():
                pltpu.make_async_copy(vv[b0], acc_sh.at[dv[b0]], ss[b0]).wait()
                pltpu.make_async_copy(vv[b1], acc_sh.at[dv[b1]], ss[b1]).wait()

            pltpu.async_copy(src_hbm.at[pl.ds(ca, CHUNK)], sv[0], se[0])
            pltpu.async_copy(dst_hbm.at[pl.ds(ca, CHUNK)], dv[b0], sd[b0])
            pltpu.async_copy(src_hbm.at[pl.ds(cb, CHUNK)], sv[1], se[1])
            pltpu.async_copy(dst_hbm.at[pl.ds(cb, CHUNK)], dv[b1], sd[b1])
            pltpu.make_async_copy(
                src_hbm.at[pl.ds(ca, CHUNK)], sv[0], se[0]).wait()
            gather(sv[0], vv[b0])
            pltpu.make_async_copy(
                src_hbm.at[pl.ds(cb, CHUNK)], sv[1], se[1]).wait()
            gather(sv[1], vv[b1])
            pltpu.make_async_copy(
                dst_hbm.at[pl.ds(ca, CHUNK)], dv[b0], sd[b0]).wait()
            pltpu.async_copy(vv[b0], acc_sh.at[dv[b0]], ss[b0], add=True)
            pltpu.make_async_copy(
                dst_hbm.at[pl.ds(cb, CHUNK)], dv[b1], sd[b1]).wait()
            pltpu.async_copy(vv[b1], acc_sh.at[dv[b1]], ss[b1], add=True)
        return carry

    lax.fori_loop(0, nquads, quad, 0)
    for b in range(4):
        pltpu.make_async_copy(vv[b], acc_sh.at[dv[b]], ss[b]).wait()


def _sc_deg(dst_hbm, out_hbm, deg_sh, dv0, dv1, dv2, dv3, onesv, zbuf,
            d0, d1, d2, d3, s0, s1, s2, s3):
    cid = lax.axis_index("c")
    sid = lax.axis_index("s")
    wid = cid * NS + sid
    sl = pl.ds(sid * SLICE, SLICE)
    _fill(onesv, CHUNK, 1.0)
    _fill(zbuf, SLICE, 0.0)
    pltpu.sync_copy(zbuf, deg_sh.at[sl])
    plsc.subcore_barrier()
    base = wid * PW
    dv = (dv0, dv1, dv2, dv3)
    sd = (d0, d1, d2, d3)
    ss = (s0, s1, s2, s3)

    def quad(t, carry):
        for half in range(2):
            b0, b1 = 2 * half, 2 * half + 1
            ca = base + (4 * t + 2 * half) * CHUNK
            cb = ca + CHUNK

            @pl.when(t > 0)
            def _():
                pltpu.make_async_copy(
                    onesv, deg_sh.at[dv[b0]], ss[b0]).wait()
                pltpu.make_async_copy(
                    onesv, deg_sh.at[dv[b1]], ss[b1]).wait()

            pltpu.async_copy(dst_hbm.at[pl.ds(ca, CHUNK)], dv[b0], sd[b0])
            pltpu.async_copy(dst_hbm.at[pl.ds(cb, CHUNK)], dv[b1], sd[b1])
            pltpu.make_async_copy(
                dst_hbm.at[pl.ds(ca, CHUNK)], dv[b0], sd[b0]).wait()
            pltpu.async_copy(onesv, deg_sh.at[dv[b0]], ss[b0], add=True)
            pltpu.make_async_copy(
                dst_hbm.at[pl.ds(cb, CHUNK)], dv[b1], sd[b1]).wait()
            pltpu.async_copy(onesv, deg_sh.at[dv[b1]], ss[b1], add=True)
        return carry

    lax.fori_loop(0, NQ_U, quad, 0)
    for b in range(4):
        pltpu.make_async_copy(onesv, deg_sh.at[dv[b]], ss[b]).wait()
    plsc.subcore_barrier()
    pltpu.sync_copy(deg_sh.at[sl], zbuf)
    pltpu.sync_copy(zbuf, out_hbm.at[cid, sl])


def _sc_u(src_hbm, dst_hbm, p_hbm, out_hbm, u_sh, p_tile,
          sv0, sv1, dv0, dv1, dv2, dv3, vv0, vv1, vv2, vv3,
          e0, e1, d0, d1, d2, d3, s0, s1, s2, s3):
    cid = lax.axis_index("c")
    sid = lax.axis_index("s")
    wid = cid * NS + sid
    off = sid * SLICE
    _zero_shared(u_sh, off, vv0)
    pltpu.sync_copy(p_hbm, p_tile)
    plsc.subcore_barrier()
    _pump(src_hbm, dst_hbm, u_sh, p_tile,
          (sv0, sv1), (dv0, dv1, dv2, dv3), (vv0, vv1, vv2, vv3),
          (e0, e1), (d0, d1, d2, d3), (s0, s1, s2, s3),
          wid * PW, NQ_U)
    plsc.subcore_barrier()
    _export_shared(u_sh, off, vv0, out_hbm.at[cid])


def _sc_v(src_hbm, dst_hbm, qp_hbm, out_hbm, v0_sh, v1_sh, qp_tile,
          sv0, sv1, dv0, dv1, va00, va01, va10, va11,
          e0, e1, d0, d1, s00, s01, s10, s11):
    cid = lax.axis_index("c")
    sid = lax.axis_index("s")
    wid = cid * NS + sid
    off = sid * SLICE
    _zero_shared(v0_sh, off, va00)
    _zero_shared(v1_sh, off, va00)
    pltpu.sync_copy(qp_hbm, qp_tile)
    plsc.subcore_barrier()
    base = wid * PW
    hi16 = jnp.full((16,), 16, _i32)
    mhi = jnp.full((16,), -65536, _i32)

    def gather(srcv, vaf0, vaf1):
        def g(i, carry):
            for u in range(5):
                o = i * 80 + u * 16
                idx = srcv[pl.ds(o, 16)]
                w = plsc.load_gather(qp_tile, [idx])
                vaf0[pl.ds(o, 16)] = lax.bitcast_convert_type(
                    lax.shift_left(w, hi16), _f32)
                vaf1[pl.ds(o, 16)] = lax.bitcast_convert_type(
                    jnp.bitwise_and(w, mhi), _f32)
            return carry

        lax.fori_loop(0, CHUNK // 80, g, 0)

    def it(t, carry):
        ca = base + (2 * t) * CHUNK
        cb = ca + CHUNK

        @pl.when(t > 0)
        def _():
            pltpu.make_async_copy(va00, v0_sh.at[dv0], s00).wait()
            pltpu.make_async_copy(va01, v1_sh.at[dv0], s01).wait()
            pltpu.make_async_copy(va10, v0_sh.at[dv1], s10).wait()
            pltpu.make_async_copy(va11, v1_sh.at[dv1], s11).wait()

        pltpu.async_copy(src_hbm.at[pl.ds(ca, CHUNK)], sv0, e0)
        pltpu.async_copy(dst_hbm.at[pl.ds(ca, CHUNK)], dv0, d0)
        pltpu.async_copy(src_hbm.at[pl.ds(cb, CHUNK)], sv1, e1)
        pltpu.async_copy(dst_hbm.at[pl.ds(cb, CHUNK)], dv1, d1)
        pltpu.make_async_copy(src_hbm.at[pl.ds(ca, CHUNK)], sv0, e0).wait()
        gather(sv0, va00, va01)
        pltpu.make_async_copy(dst_hbm.at[pl.ds(ca, CHUNK)], dv0, d0).wait()
        pltpu.async_copy(va00, v0_sh.at[dv0], s00, add=True)
        pltpu.async_copy(va01, v1_sh.at[dv0], s01, add=True)
        pltpu.make_async_copy(src_hbm.at[pl.ds(cb, CHUNK)], sv1, e1).wait()
        gather(sv1, va10, va11)
        pltpu.make_async_copy(dst_hbm.at[pl.ds(cb, CHUNK)], dv1, d1).wait()
        pltpu.async_copy(va10, v0_sh.at[dv1], s10, add=True)
        pltpu.async_copy(va11, v1_sh.at[dv1], s11, add=True)
        return carry

    lax.fori_loop(0, NQ_U * 2, it, 0)
    pltpu.make_async_copy(va00, v0_sh.at[dv0], s00).wait()
    pltpu.make_async_copy(va01, v1_sh.at[dv0], s01).wait()
    pltpu.make_async_copy(va10, v0_sh.at[dv1], s10).wait()
    pltpu.make_async_copy(va11, v1_sh.at[dv1], s11).wait()
    plsc.subcore_barrier()
    _export_shared(v0_sh, off, va00, out_hbm.at[cid, 0])
    _export_shared(v1_sh, off, va00, out_hbm.at[cid, 1])


def _tc_a(degp_ref, x_ref, dis_ref, p_ref):
    deg = degp_ref[0] + degp_ref[1] + 1.0
    dis = lax.rsqrt(deg)
    dis_ref[...] = dis
    p_ref[...] = dis * x_ref[...]


def _tc_b(up_ref, dis_ref, x_ref, w1_ref, b1_ref, w2_ref, b2_ref,
          qp_ref, base0_ref, base1_ref):
    dis = dis_ref[...]
    dis2 = dis * dis
    s = dis * (up_ref[0] + up_ref[1]) + dis2 * x_ref[...]
    acc0 = jnp.zeros_like(s)
    acc1 = jnp.zeros_like(s)
    for k in range(8):
        h = jnp.maximum(s * w1_ref[0, k] + b1_ref[k], 0.0)
        acc0 = acc0 + h * w2_ref[k, 0]
        acc1 = acc1 + h * w2_ref[k, 1]
    r0 = lax.bitcast_convert_type((dis * acc0).astype(jnp.bfloat16),
                                  jnp.uint16)
    r1 = lax.bitcast_convert_type((dis * acc1).astype(jnp.bfloat16),
                                  jnp.uint16)
    packed = lax.shift_left(r1.astype(jnp.uint32), jnp.uint32(16)) | \
        r0.astype(jnp.uint32)
    qp_ref[...] = lax.bitcast_convert_type(packed, _i32)
    base0_ref[...] = dis2 * acc0 + b2_ref[0]
    base1_ref[...] = dis2 * acc1 + b2_ref[1]


def _tc_c(v_ref, dis_ref, base0_ref, base1_ref, o0_ref, o1_ref):
    dis = dis_ref[...]
    o0_ref[...] = dis * (v_ref[0, 0] + v_ref[1, 0]) + base0_ref[...]
    o1_ref[...] = dis * (v_ref[0, 1] + v_ref[1, 1]) + base1_ref[...]


_nodes = jax.ShapeDtypeStruct((R, 128), _f32)
_DMA = pltpu.SemaphoreType.DMA
_SC_PARAMS = pltpu.CompilerParams(needs_layout_passes=False)
_IDX4 = [pltpu.VMEM((CHUNK,), _i32)] * 4
_IDX2 = [pltpu.VMEM((CHUNK,), _i32)] * 2
_VAL4 = [pltpu.VMEM((CHUNK,), _f32)] * 4


def kernel(x, edge_index, W1, b1, W2, b2):
    ei = edge_index.astype(_i32)
    src = ei[0]
    dst = ei[1]
    xp = jnp.pad(x[:, 0], (0, NPAD - N))

    deg_p = pl.kernel(
        _sc_deg,
        out_type=jax.ShapeDtypeStruct((NC, NPAD), _f32),
        mesh=_mesh,
        compiler_params=_SC_PARAMS,
        scratch_types=[
            pltpu.VMEM_SHARED((NPAD,), _f32),
            *_IDX4,
            pltpu.VMEM((CHUNK,), _f32),
            pltpu.VMEM((SLICE,), _f32),
            *([_DMA] * 8),
        ],
    )(dst)

    dis, p = pl.pallas_call(
        _tc_a,
        out_shape=(_nodes, _nodes),
    )(deg_p.reshape(NC, R, 128), xp.reshape(R, 128))

    u_p = pl.kernel(
        _sc_u,
        out_type=jax.ShapeDtypeStruct((NC, NPAD), _f32),
        mesh=_mesh,
        compiler_params=_SC_PARAMS,
        scratch_types=[
            pltpu.VMEM_SHARED((NPAD,), _f32),
            pltpu.VMEM((NPAD,), _f32),
            *_IDX2, *_IDX4, *_VAL4,
            *([_DMA] * 10),
        ],
    )(src, dst, p.reshape(NPAD))

    smem = pl.BlockSpec(memory_space=pltpu.SMEM)
    vmem = pl.BlockSpec(memory_space=pltpu.VMEM)
    qp, base0, base1 = pl.pallas_call(
        _tc_b,
        out_shape=(jax.ShapeDtypeStruct((R, 128), _i32), _nodes, _nodes),
        in_specs=[vmem, vmem, vmem, smem, smem, smem, smem],
    )(u_p.reshape(NC, R, 128), dis, xp.reshape(R, 128), W1, b1, W2, b2)

    v_p = pl.kernel(
        _sc_v,
        out_type=jax.ShapeDtypeStruct((NC, 2, NPAD), _f32),
        mesh=_mesh,
        compiler_params=_SC_PARAMS,
        scratch_types=[
            pltpu.VMEM_SHARED((NPAD,), _f32),
            pltpu.VMEM_SHARED((NPAD,), _f32),
            pltpu.VMEM((NPAD,), _i32),
            *_IDX2, *_IDX2, *_VAL4,
            *([_DMA] * 8),
        ],
    )(src, dst, qp.reshape(NPAD))

    o0, o1 = pl.pallas_call(
        _tc_c,
        out_shape=(_nodes, _nodes),
    )(v_p.reshape(NC, 2, R, 128), dis, base0, base1)

    return jnp.stack([o0.reshape(NPAD)[:N], o1.reshape(NPAD)[:N]], axis=1)


# speedup vs baseline: 1.0123x; 1.0123x over previous
import jax
import jax.numpy as jnp
from jax import lax
from jax.experimental import pallas as pl
from jax.experimental.pallas import tpu as pltpu
from jax.experimental.pallas import tpu_sc as plsc

N = 100000
E = 6400000
NPAD = 100352
R = NPAD // 128
NC, NS = 2, 16
NW = NC * NS
SLICE = NPAD // NS
CHUNK = 2000
NQ_U = E // (NW * CHUNK) // 4
NQ_V = E // (NS * CHUNK) // 4
PW = E // NW
PWV = E // NS

_mesh = plsc.VectorSubcoreMesh(core_axis_name="c", subcore_axis_name="s")
_f32 = jnp.float32
_i32 = jnp.int32
_PC = 1920
_PIECES = [(o, min(_PC, SLICE - o)) for o in range(0, SLICE, _PC)]


def _fill(buf, nwords, value):
    vec = jnp.full((16,), value, _f32)

    def body(i, carry):
        buf[pl.ds(i * 16, 16)] = vec
        return carry

    lax.fori_loop(0, nwords // 16, body, 0)


def _zero_shared(sh_ref, off, valv):
    _fill(valv, CHUNK, 0.0)
    for o, ln in _PIECES:
        pltpu.sync_copy(valv.at[pl.ds(0, ln)], sh_ref.at[pl.ds(off + o, ln)])


def _export_shared(sh_ref, off, valv, out_slot):
    for o, ln in _PIECES:
        pltpu.sync_copy(sh_ref.at[pl.ds(off + o, ln)], valv.at[pl.ds(0, ln)])
        pltpu.sync_copy(valv.at[pl.ds(0, ln)], out_slot.at[pl.ds(off + o, ln)])


def _pump(src_hbm, dst_hbm, acc_sh, table, sv, dv, vv, se, sd, ss,
          base, nquads):
    def gather(srcv, valv):
        def g(i, carry):
            for u in range(5):
                o = i * 80 + u * 16
                idx = srcv[pl.ds(o, 16)]
                valv[pl.ds(o, 16)] = plsc.load_gather(table, [idx])
            return carry

        lax.fori_loop(0, CHUNK // 80, g, 0)

    def quad(t, carry):
        for half in range(2):
            b0, b1 = 2 * half, 2 * half + 1
            ca = base + (4 * t + 2 * half) * CHUNK
            cb = ca + CHUNK

            @pl.when(t > 0)
            def _():
                pltpu.make_async_copy(vv[b0], acc_sh.at[dv[b0]], ss[b0]).wait()
                pltpu.make_async_copy(vv[b1], acc_sh.at[dv[b1]], ss[b1]).wait()

            pltpu.async_copy(src_hbm.at[pl.ds(ca, CHUNK)], sv[0], se[0])
            pltpu.async_copy(dst_hbm.at[pl.ds(ca, CHUNK)], dv[b0], sd[b0])
            pltpu.async_copy(src_hbm.at[pl.ds(cb, CHUNK)], sv[1], se[1])
            pltpu.async_copy(dst_hbm.at[pl.ds(cb, CHUNK)], dv[b1], sd[b1])
            pltpu.make_async_copy(
                src_hbm.at[pl.ds(ca, CHUNK)], sv[0], se[0]).wait()
            gather(sv[0], vv[b0])
            pltpu.make_async_copy(
                src_hbm.at[pl.ds(cb, CHUNK)], sv[1], se[1]).wait()
            gather(sv[1], vv[b1])
            pltpu.make_async_copy(
                dst_hbm.at[pl.ds(ca, CHUNK)], dv[b0], sd[b0]).wait()
            pltpu.async_copy(vv[b0], acc_sh.at[dv[b0]], ss[b0], add=True)
            pltpu.make_async_copy(
                dst_hbm.at[pl.ds(cb, CHUNK)], dv[b1], sd[b1]).wait()
            pltpu.async_copy(vv[b1], acc_sh.at[dv[b1]], ss[b1], add=True)
        return carry

    lax.fori_loop(0, nquads, quad, 0)
    for b in range(4):
        pltpu.make_async_copy(vv[b], acc_sh.at[dv[b]], ss[b]).wait()


def _sc_deg(dst_hbm, out_hbm, deg_sh, dv0, dv1, dv2, dv3, onesv, zbuf,
            d0, d1, d2, d3, s0, s1, s2, s3):
    cid = lax.axis_index("c")
    sid = lax.axis_index("s")
    wid = cid * NS + sid
    sl = pl.ds(sid * SLICE, SLICE)
    _fill(onesv, CHUNK, 1.0)
    _fill(zbuf, SLICE, 0.0)
    pltpu.sync_copy(zbuf, deg_sh.at[sl])
    plsc.subcore_barrier()
    base = wid * PW
    dv = (dv0, dv1, dv2, dv3)
    sd = (d0, d1, d2, d3)
    ss = (s0, s1, s2, s3)

    def quad(t, carry):
        for half in range(2):
            b0, b1 = 2 * half, 2 * half + 1
            ca = base + (4 * t + 2 * half) * CHUNK
            cb = ca + CHUNK

            @pl.when(t > 0)
            def _():
                pltpu.make_async_copy(
                    onesv, deg_sh.at[dv[b0]], ss[b0]).wait()
                pltpu.make_async_copy(
                    onesv, deg_sh.at[dv[b1]], ss[b1]).wait()

            pltpu.async_copy(dst_hbm.at[pl.ds(ca, CHUNK)], dv[b0], sd[b0])
            pltpu.async_copy(dst_hbm.at[pl.ds(cb, CHUNK)], dv[b1], sd[b1])
            pltpu.make_async_copy(
                dst_hbm.at[pl.ds(ca, CHUNK)], dv[b0], sd[b0]).wait()
            pltpu.async_copy(onesv, deg_sh.at[dv[b0]], ss[b0], add=True)
            pltpu.make_async_copy(
                dst_hbm.at[pl.ds(cb, CHUNK)], dv[b1], sd[b1]).wait()
            pltpu.async_copy(onesv, deg_sh.at[dv[b1]], ss[b1], add=True)
        return carry

    lax.fori_loop(0, NQ_U, quad, 0)
    for b in range(4):
        pltpu.make_async_copy(onesv, deg_sh.at[dv[b]], ss[b]).wait()
    plsc.subcore_barrier()
    pltpu.sync_copy(deg_sh.at[sl], zbuf)
    pltpu.sync_copy(zbuf, out_hbm.at[cid, sl])


def _sc_u(src_hbm, dst_hbm, p_hbm, out_hbm, u_sh, p_tile,
          sv0, sv1, dv0, dv1, dv2, dv3, vv0, vv1, vv2, vv3,
          e0, e1, d0, d1, d2, d3, s0, s1, s2, s3):
    cid = lax.axis_index("c")
    sid = lax.axis_index("s")
    wid = cid * NS + sid
    off = sid * SLICE
    _zero_shared(u_sh, off, vv0)
    pltpu.sync_copy(p_hbm, p_tile)
    plsc.subcore_barrier()
    _pump(src_hbm, dst_hbm, u_sh, p_tile,
          (sv0, sv1), (dv0, dv1, dv2, dv3), (vv0, vv1, vv2, vv3),
          (e0, e1), (d0, d1, d2, d3), (s0, s1, s2, s3),
          wid * PW, NQ_U)
    plsc.subcore_barrier()
    _export_shared(u_sh, off, vv0, out_hbm.at[cid])


def _sc_v(src_hbm, dst_hbm, q0_hbm, q1_hbm, out_hbm, v_sh, q_tile,
          sv0, sv1, dv0, dv1, dv2, dv3, vv0, vv1, vv2, vv3,
          e0, e1, d0, d1, d2, d3, s0, s1, s2, s3):
    cid = lax.axis_index("c")
    sid = lax.axis_index("s")
    off = sid * SLICE
    _zero_shared(v_sh, off, vv0)

    @pl.when(cid == 0)
    def _():
        pltpu.sync_copy(q0_hbm, q_tile)

    @pl.when(cid == 1)
    def _():
        pltpu.sync_copy(q1_hbm, q_tile)

    plsc.subcore_barrier()
    _pump(src_hbm, dst_hbm, v_sh, q_tile,
          (sv0, sv1), (dv0, dv1, dv2, dv3), (vv0, vv1, vv2, vv3),
          (e0, e1), (d0, d1, d2, d3), (s0, s1, s2, s3),
          sid * PWV, NQ_V)
    plsc.subcore_barrier()
    _export_shared(v_sh, off, vv0, out_hbm.at[cid])


def _tc_a(degp_ref, x_ref, dis_ref, p_ref):
    deg = degp_ref[0] + degp_ref[1] + 1.0
    dis = lax.rsqrt(deg)
    dis_ref[...] = dis
    p_ref[...] = dis * x_ref[...]


def _tc_b(up_ref, dis_ref, x_ref, w1_ref, b1_ref, w2_ref, b2_ref,
          q0_ref, q1_ref, base0_ref, base1_ref):
    dis = dis_ref[...]
    dis2 = dis * dis
    s = dis * (up_ref[0] + up_ref[1]) + dis2 * x_ref[...]
    acc0 = jnp.zeros_like(s)
    acc1 = jnp.zeros_like(s)
    for k in range(8):
        h = jnp.maximum(s * w1_ref[0, k] + b1_ref[k], 0.0)
        acc0 = acc0 + h * w2_ref[k, 0]
        acc1 = acc1 + h * w2_ref[k, 1]
    q0_ref[...] = dis * acc0
    q1_ref[...] = dis * acc1
    base0_ref[...] = dis2 * acc0 + b2_ref[0]
    base1_ref[...] = dis2 * acc1 + b2_ref[1]


def _tc_c(v_ref, dis_ref, base0_ref, base1_ref, o0_ref, o1_ref):
    dis = dis_ref[...]
    o0_ref[...] = dis * v_ref[0] + base0_ref[...]
    o1_ref[...] = dis * v_ref[1] + base1_ref[...]


_nodes = jax.ShapeDtypeStruct((R, 128), _f32)
_DMA = pltpu.SemaphoreType.DMA
_SC_PARAMS = pltpu.CompilerParams(needs_layout_passes=False)
_IDX4 = [pltpu.VMEM((CHUNK,), _i32)] * 4
_IDX2 = [pltpu.VMEM((CHUNK,), _i32)] * 2
_VAL4 = [pltpu.VMEM((CHUNK,), _f32)] * 4


def kernel(x, edge_index, W1, b1, W2, b2):
    ei = edge_index.astype(_i32)
    src = ei[0]
    dst = ei[1]
    xp = jnp.pad(x[:, 0], (0, NPAD - N))

    deg_p = pl.kernel(
        _sc_deg,
        out_type=jax.ShapeDtypeStruct((NC, NPAD), _f32),
        mesh=_mesh,
        compiler_params=_SC_PARAMS,
        scratch_types=[
            pltpu.VMEM_SHARED((NPAD,), _f32),
            *_IDX4,
            pltpu.VMEM((CHUNK,), _f32),
            pltpu.VMEM((SLICE,), _f32),
            *([_DMA] * 8),
        ],
    )(dst)

    dis, p = pl.pallas_call(
        _tc_a,
        out_shape=(_nodes, _nodes),
    )(deg_p.reshape(NC, R, 128), xp.reshape(R, 128))

    u_p = pl.kernel(
        _sc_u,
        out_type=jax.ShapeDtypeStruct((NC, NPAD), _f32),
        mesh=_mesh,
        compiler_params=_SC_PARAMS,
        scratch_types=[
            pltpu.VMEM_SHARED((NPAD,), _f32),
            pltpu.VMEM((NPAD,), _f32),
            *_IDX2, *_IDX4, *_VAL4,
            *([_DMA] * 10),
        ],
    )(src, dst, p.reshape(NPAD))

    smem = pl.BlockSpec(memory_space=pltpu.SMEM)
    vmem = pl.BlockSpec(memory_space=pltpu.VMEM)
    q0, q1, base0, base1 = pl.pallas_call(
        _tc_b,
        out_shape=(_nodes, _nodes, _nodes, _nodes),
        in_specs=[vmem, vmem, vmem, smem, smem, smem, smem],
    )(u_p.reshape(NC, R, 128), dis, xp.reshape(R, 128), W1, b1, W2, b2)

    v_out = pl.kernel(
        _sc_v,
        out_type=jax.ShapeDtypeStruct((NC, NPAD), _f32),
        mesh=_mesh,
        compiler_params=_SC_PARAMS,
        scratch_types=[
            pltpu.VMEM_SHARED((NPAD,), _f32),
            pltpu.VMEM((NPAD,), _f32),
            *_IDX2, *_IDX4, *_VAL4,
            *([_DMA] * 10),
        ],
    )(src, dst, q0.reshape(NPAD), q1.reshape(NPAD))

    o0, o1 = pl.pallas_call(
        _tc_c,
        out_shape=(_nodes, _nodes),
    )(v_out.reshape(NC, R, 128), dis, base0, base1)

    return jnp.stack([o0.reshape(NPAD)[:N], o1.reshape(NPAD)[:N]], axis=1)
